# initial kernel scaffold (unmeasured)
import jax
import jax.numpy as jnp
from jax import lax
from jax.experimental import pallas as pl
from jax.experimental.pallas import tpu as pltpu

N_DEV = 4
SQ = 256
D = 1024
H = 8
DH = 128
SCALE = 0.08838834764831843


def kernel(x, Wq, Wo, Wk, Wv):
    def body(x_ref, wq_ref, wo_ref, wk_ref, wv_ref, out_ref,
             xs_ref, part_ref, rs_send_ref, rs_recv_ref,
             ag_send_sems, ag_recv_sems, rs_send_sems, rs_recv_sems):
        my = lax.axis_index("i")
        right = lax.rem(my + 1, N_DEV)
        left = lax.rem(my + N_DEV - 1, N_DEV)

        barrier_sem = pltpu.get_barrier_semaphore()
        for nbr in (left, right):
            pl.semaphore_signal(
                barrier_sem, inc=1,
                device_id=(nbr,), device_id_type=pl.DeviceIdType.MESH,
            )
        pl.semaphore_wait(barrier_sem, 2)

        xs_ref[0, :, :] = x_ref[0, :, :].astype(jnp.bfloat16)

        for h in range(N_DEV - 1):
            rdma = pltpu.make_async_remote_copy(
                src_ref=xs_ref.at[h],
                dst_ref=xs_ref.at[h + 1],
                send_sem=ag_send_sems.at[h],
                recv_sem=ag_recv_sems.at[h],
                device_id=(right,),
                device_id_type=pl.DeviceIdType.MESH,
            )
            rdma.start()
            rdma.wait()

        wq = wq_ref[:, :].astype(jnp.bfloat16)
        wk = wk_ref[:, :].astype(jnp.bfloat16)
        wv = wv_ref[:, :].astype(jnp.bfloat16)
        wo = wo_ref[:, :].astype(jnp.bfloat16)

        def partial_for(xb):
            q = lax.dot_general(xb, wq, (((1,), (0,)), ((), ())),
                                preferred_element_type=jnp.bfloat16)
            k = lax.dot_general(xb, wk, (((1,), (0,)), ((), ())),
                                preferred_element_type=jnp.bfloat16)
            v = lax.dot_general(xb, wv, (((1,), (0,)), ((), ())),
                                preferred_element_type=jnp.bfloat16)
            cols = []
            for h in range(H):
                sl = slice(h * DH, (h + 1) * DH)
                s = lax.dot_general(q[:, sl], k[:, sl],
                                    (((1,), (1,)), ((), ())),
                                    preferred_element_type=jnp.float32)
                s = s * SCALE
                m = jnp.max(s, axis=1, keepdims=True)
                p = jnp.exp(s - m)
                p = p / jnp.sum(p, axis=1, keepdims=True)
                o = lax.dot_general(p.astype(jnp.bfloat16), v[:, sl],
                                    (((1,), (0,)), ((), ())),
                                    preferred_element_type=jnp.float32)
                cols.append(o)
            attn = jnp.concatenate(cols, axis=1).astype(jnp.bfloat16)
            return lax.dot_general(attn, wo, (((1,), (0,)), ((), ())),
                                   preferred_element_type=jnp.float32)

        for k in range(N_DEV):
            part_ref[k, :, :] = partial_for(xs_ref[k, :, :])

        for s in range(N_DEV - 1):
            if s == 0:
                acc = part_ref[1, :, :]
            else:
                acc = part_ref[s + 1, :, :] + rs_recv_ref[s - 1, :, :].astype(jnp.float32)
            rs_send_ref[s, :, :] = acc.astype(jnp.bfloat16)
            rdma = pltpu.make_async_remote_copy(
                src_ref=rs_send_ref.at[s],
                dst_ref=rs_recv_ref.at[s],
                send_sem=rs_send_sems.at[s],
                recv_sem=rs_recv_sems.at[s],
                device_id=(right,),
                device_id_type=pl.DeviceIdType.MESH,
            )
            rdma.start()
            rdma.wait()

        out_ref[0, :, :] = part_ref[0, :, :] + rs_recv_ref[N_DEV - 2, :, :].astype(jnp.float32)

    return pl.pallas_call(
        body,
        out_shape=jax.ShapeDtypeStruct((1, SQ, D), jnp.float32),
        in_specs=[pl.BlockSpec(memory_space=pltpu.VMEM)] * 5,
        out_specs=pl.BlockSpec(memory_space=pltpu.VMEM),
        scratch_shapes=[
            pltpu.VMEM((N_DEV, SQ, D), jnp.bfloat16),
            pltpu.VMEM((N_DEV, SQ, D), jnp.float32),
            pltpu.VMEM((N_DEV - 1, SQ, D), jnp.bfloat16),
            pltpu.VMEM((N_DEV - 1, SQ, D), jnp.bfloat16),
            pltpu.SemaphoreType.DMA((N_DEV - 1,)),
            pltpu.SemaphoreType.DMA((N_DEV - 1,)),
            pltpu.SemaphoreType.DMA((N_DEV - 1,)),
            pltpu.SemaphoreType.DMA((N_DEV - 1,)),
        ],
        compiler_params=pltpu.CompilerParams(collective_id=0),
    )(x, Wq, Wo, Wk, Wv)


# baseline (device time: 72141 ns/iter reference)
import jax
import jax.numpy as jnp
from jax import lax
from jax.experimental import pallas as pl
from jax.experimental.pallas import tpu as pltpu

N_DEV = 4
SQ = 256
D = 1024
H = 8
DH = 128
SCALE = 0.08838834764831843


def kernel(x, Wq, Wo, Wk, Wv):
    def body(x_ref, wq_ref, wo_ref, wk_ref, wv_ref, out_ref,
             xs_ref, part_ref, rs_send_ref, rs_recv_ref,
             ag_send_sems, ag_recv_sems, rs_send_sems, rs_recv_sems):
        my = lax.axis_index("i")
        right = lax.rem(my + 1, N_DEV)
        left = lax.rem(my + N_DEV - 1, N_DEV)

        barrier_sem = pltpu.get_barrier_semaphore()
        for nbr in (left, right):
            pl.semaphore_signal(
                barrier_sem, inc=1,
                device_id=(nbr,), device_id_type=pl.DeviceIdType.MESH,
            )
        pl.semaphore_wait(barrier_sem, 2)

        xs_ref[0, :, :] = x_ref[0, :, :].astype(jnp.bfloat16)

        for h in range(N_DEV - 1):
            rdma = pltpu.make_async_remote_copy(
                src_ref=xs_ref.at[h],
                dst_ref=xs_ref.at[h + 1],
                send_sem=ag_send_sems.at[h],
                recv_sem=ag_recv_sems.at[h],
                device_id=(right,),
                device_id_type=pl.DeviceIdType.MESH,
            )
            rdma.start()
            rdma.wait()

        wq = wq_ref[:, :].astype(jnp.bfloat16)
        wk = wk_ref[:, :].astype(jnp.bfloat16)
        wv = wv_ref[:, :].astype(jnp.bfloat16)
        wo = wo_ref[:, :].astype(jnp.bfloat16)

        def partial_for(xb):
            q = lax.dot_general(xb, wq, (((1,), (0,)), ((), ())),
                                preferred_element_type=jnp.float32).astype(jnp.bfloat16)
            k = lax.dot_general(xb, wk, (((1,), (0,)), ((), ())),
                                preferred_element_type=jnp.float32).astype(jnp.bfloat16)
            v = lax.dot_general(xb, wv, (((1,), (0,)), ((), ())),
                                preferred_element_type=jnp.float32).astype(jnp.bfloat16)
            cols = []
            for h in range(H):
                sl = slice(h * DH, (h + 1) * DH)
                s = lax.dot_general(q[:, sl], k[:, sl],
                                    (((1,), (1,)), ((), ())),
                                    preferred_element_type=jnp.float32)
                s = s * SCALE
                m = jnp.max(s, axis=1, keepdims=True)
                p = jnp.exp(s - m)
                p = p / jnp.sum(p, axis=1, keepdims=True)
                o = lax.dot_general(p.astype(jnp.bfloat16), v[:, sl],
                                    (((1,), (0,)), ((), ())),
                                    preferred_element_type=jnp.float32)
                cols.append(o)
            attn = jnp.concatenate(cols, axis=1).astype(jnp.bfloat16)
            return lax.dot_general(attn, wo, (((1,), (0,)), ((), ())),
                                   preferred_element_type=jnp.float32)

        for k in range(N_DEV):
            part_ref[k, :, :] = partial_for(xs_ref[k, :, :])

        for s in range(N_DEV - 1):
            if s == 0:
                acc = part_ref[1, :, :]
            else:
                acc = part_ref[s + 1, :, :] + rs_recv_ref[s - 1, :, :].astype(jnp.float32)
            rs_send_ref[s, :, :] = acc.astype(jnp.bfloat16)
            rdma = pltpu.make_async_remote_copy(
                src_ref=rs_send_ref.at[s],
                dst_ref=rs_recv_ref.at[s],
                send_sem=rs_send_sems.at[s],
                recv_sem=rs_recv_sems.at[s],
                device_id=(right,),
                device_id_type=pl.DeviceIdType.MESH,
            )
            rdma.start()
            rdma.wait()

        out_ref[0, :, :] = part_ref[0, :, :] + rs_recv_ref[N_DEV - 2, :, :].astype(jnp.float32)

    return pl.pallas_call(
        body,
        out_shape=jax.ShapeDtypeStruct((1, SQ, D), jnp.float32),
        in_specs=[pl.BlockSpec(memory_space=pltpu.VMEM)] * 5,
        out_specs=pl.BlockSpec(memory_space=pltpu.VMEM),
        scratch_shapes=[
            pltpu.VMEM((N_DEV, SQ, D), jnp.bfloat16),
            pltpu.VMEM((N_DEV, SQ, D), jnp.float32),
            pltpu.VMEM((N_DEV - 1, SQ, D), jnp.bfloat16),
            pltpu.VMEM((N_DEV - 1, SQ, D), jnp.bfloat16),
            pltpu.SemaphoreType.DMA((N_DEV - 1,)),
            pltpu.SemaphoreType.DMA((N_DEV - 1,)),
            pltpu.SemaphoreType.DMA((N_DEV - 1,)),
            pltpu.SemaphoreType.DMA((N_DEV - 1,)),
        ],
        compiler_params=pltpu.CompilerParams(collective_id=0),
    )(x, Wq, Wo, Wk, Wv)


# device time: 44833 ns/iter; 1.6091x vs baseline; 1.6091x over previous
import jax
import jax.numpy as jnp
from jax import lax
from jax.experimental import pallas as pl
from jax.experimental.pallas import tpu as pltpu

N_DEV = 4
SQ = 256
HALF = SQ // 2
D = 1024
H = 8
DH = 128
SCALE = 0.08838834764831843

AGR0, AGR1, AGL0, RSL1A, RSL1B, RSL2A, RSL2B, RSRA, RSRB = range(9)


def kernel(x, Wq, Wo, Wk, Wv):
    def body(x_ref, wq_ref, wo_ref, wk_ref, wv_ref, out_ref,
             xs_ref, pown_ref, pm1_ref,
             rsr_s, rsr_r, rsl1_s, rsl1_r, rsl2_s, rsl2_r,
             send_sems, recv_sems):
        my = lax.axis_index("i")
        right = lax.rem(my + 1, N_DEV)
        left = lax.rem(my + N_DEV - 1, N_DEV)

        barrier_sem = pltpu.get_barrier_semaphore()
        for nbr in (left, right):
            pl.semaphore_signal(
                barrier_sem, inc=1,
                device_id=(nbr,), device_id_type=pl.DeviceIdType.MESH,
            )
        pl.semaphore_wait(barrier_sem, 2)

        def copy(src, dst, idx, dev):
            return pltpu.make_async_remote_copy(
                src_ref=src, dst_ref=dst,
                send_sem=send_sems.at[idx], recv_sem=recv_sems.at[idx],
                device_id=(dev,), device_id_type=pl.DeviceIdType.MESH,
            )

        xs_ref[0, :, :] = x_ref[0, :, :].astype(jnp.bfloat16)
        agr0 = copy(xs_ref.at[0], xs_ref.at[1], AGR0, right)
        agl0 = copy(xs_ref.at[0], xs_ref.at[3], AGL0, left)
        agr0.start()
        agl0.start()

        wq = wq_ref[:, :].astype(jnp.bfloat16)
        wk = wk_ref[:, :].astype(jnp.bfloat16)
        wv = wv_ref[:, :].astype(jnp.bfloat16)
        wo = wo_ref[:, :].astype(jnp.bfloat16)

        def kv_for(xb):
            k = lax.dot_general(xb, wk, (((1,), (0,)), ((), ())),
                                preferred_element_type=jnp.float32).astype(jnp.bfloat16)
            v = lax.dot_general(xb, wv, (((1,), (0,)), ((), ())),
                                preferred_element_type=jnp.float32).astype(jnp.bfloat16)
            return k, v

        def p_rows(xb_rows, k, v):
            q = lax.dot_general(xb_rows, wq, (((1,), (0,)), ((), ())),
                                preferred_element_type=jnp.float32).astype(jnp.bfloat16)
            cols = []
            for h in range(H):
                sl = slice(h * DH, (h + 1) * DH)
                s = lax.dot_general(q[:, sl], k[:, sl],
                                    (((1,), (1,)), ((), ())),
                                    preferred_element_type=jnp.float32)
                s = s * SCALE
                m = jnp.max(s, axis=1, keepdims=True)
                p = jnp.exp(s - m)
                p = p / jnp.sum(p, axis=1, keepdims=True)
                o = lax.dot_general(p.astype(jnp.bfloat16), v[:, sl],
                                    (((1,), (0,)), ((), ())),
                                    preferred_element_type=jnp.float32)
                cols.append(o)
            attn = jnp.concatenate(cols, axis=1).astype(jnp.bfloat16)
            return lax.dot_general(attn, wo, (((1,), (0,)), ((), ())),
                                   preferred_element_type=jnp.float32)

        def partial_for(xb):
            k, v = kv_for(xb)
            return p_rows(xb, k, v)

        pown_ref[:, :] = partial_for(xs_ref[0, :, :])

        agr0.wait_recv()
        agr1 = copy(xs_ref.at[1], xs_ref.at[2], AGR1, right)
        agr1.start()

        pm1_ref[:, :] = partial_for(xs_ref[1, :, :])

        agr1.wait_recv()
        k2, v2 = kv_for(xs_ref[2, :, :])
        rsl1_s[0, :, :] = p_rows(xs_ref[2, 0:HALF, :], k2, v2).astype(jnp.bfloat16)
        rsl1a = copy(rsl1_s.at[0], rsl1_r.at[0], RSL1A, left)
        rsl1a.start()
        rsl1_s[1, :, :] = p_rows(xs_ref[2, HALF:SQ, :], k2, v2).astype(jnp.bfloat16)
        rsl1b = copy(rsl1_s.at[1], rsl1_r.at[1], RSL1B, left)
        rsl1b.start()

        agl0.wait_recv()
        k3, v3 = kv_for(xs_ref[3, :, :])
        rsr_s[0, :, :] = p_rows(xs_ref[3, 0:HALF, :], k3, v3).astype(jnp.bfloat16)
        rsra = copy(rsr_s.at[0], rsr_r.at[0], RSRA, right)
        rsra.start()

        rsl1a.wait_recv()
        rsl2_s[0, :, :] = (pm1_ref[0:HALF, :]
                           + rsl1_r[0, :, :].astype(jnp.float32)).astype(jnp.bfloat16)
        rsl2a = copy(rsl2_s.at[0], rsl2_r.at[0], RSL2A, left)
        rsl2a.start()

        rsr_s[1, :, :] = p_rows(xs_ref[3, HALF:SQ, :], k3, v3).astype(jnp.bfloat16)
        rsrb = copy(rsr_s.at[1], rsr_r.at[1], RSRB, right)
        rsrb.start()

        rsl1b.wait_recv()
        rsl2_s[1, :, :] = (pm1_ref[HALF:SQ, :]
                           + rsl1_r[1, :, :].astype(jnp.float32)).astype(jnp.bfloat16)
        rsl2b = copy(rsl2_s.at[1], rsl2_r.at[1], RSL2B, left)
        rsl2b.start()

        rsl2a.wait_recv()
        rsra.wait_recv()
        out_ref[0, 0:HALF, :] = (pown_ref[0:HALF, :]
                                 + rsl2_r[0, :, :].astype(jnp.float32)
                                 + rsr_r[0, :, :].astype(jnp.float32))
        rsl2b.wait_recv()
        rsrb.wait_recv()
        out_ref[0, HALF:SQ, :] = (pown_ref[HALF:SQ, :]
                                  + rsl2_r[1, :, :].astype(jnp.float32)
                                  + rsr_r[1, :, :].astype(jnp.float32))

        for rdma in (agr0, agl0, agr1, rsl1a, rsl1b, rsra, rsrb, rsl2a, rsl2b):
            rdma.wait_send()

    return pl.pallas_call(
        body,
        out_shape=jax.ShapeDtypeStruct((1, SQ, D), jnp.float32),
        in_specs=[pl.BlockSpec(memory_space=pltpu.VMEM)] * 5,
        out_specs=pl.BlockSpec(memory_space=pltpu.VMEM),
        scratch_shapes=[
            pltpu.VMEM((N_DEV, SQ, D), jnp.bfloat16),
            pltpu.VMEM((SQ, D), jnp.float32),
            pltpu.VMEM((SQ, D), jnp.float32),
            pltpu.VMEM((2, HALF, D), jnp.bfloat16),
            pltpu.VMEM((2, HALF, D), jnp.bfloat16),
            pltpu.VMEM((2, HALF, D), jnp.bfloat16),
            pltpu.VMEM((2, HALF, D), jnp.bfloat16),
            pltpu.VMEM((2, HALF, D), jnp.bfloat16),
            pltpu.VMEM((2, HALF, D), jnp.bfloat16),
            pltpu.SemaphoreType.DMA((9,)),
            pltpu.SemaphoreType.DMA((9,)),
        ],
        compiler_params=pltpu.CompilerParams(collective_id=0),
    )(x, Wq, Wo, Wk, Wv)


# device time: 41165 ns/iter; 1.7525x vs baseline; 1.0891x over previous
import jax
import jax.numpy as jnp
from jax import lax
from jax.experimental import pallas as pl
from jax.experimental.pallas import tpu as pltpu

N_DEV = 4
SQ = 256
HALF = SQ // 2
D = 1024
H = 8
DH = 128
SCALE = 0.08838834764831843

AGR0, AGR1, AGL0, PL, PDA, PDB, PRA, PRB = range(8)


def kernel(x, Wq, Wo, Wk, Wv):
    def body(x_ref, wq_ref, wo_ref, wk_ref, wv_ref, out_ref,
             x_vmem, xs_ref, pown_ref,
             pl_s, pl_r, pd_s, pd_r, pr_s, pr_r,
             w_vmem, w_sems, x_sem,
             send_sems, recv_sems):
        my = lax.axis_index("i")
        right = lax.rem(my + 1, N_DEV)
        left = lax.rem(my + N_DEV - 1, N_DEV)
        diag = lax.rem(my + 2, N_DEV)

        x_dma = pltpu.make_async_copy(x_ref, x_vmem, x_sem)
        x_dma.start()
        w_dmas = []
        for j, w_hbm in enumerate((wk_ref, wv_ref, wq_ref, wo_ref)):
            dma = pltpu.make_async_copy(w_hbm, w_vmem.at[j], w_sems.at[j])
            dma.start()
            w_dmas.append(dma)

        barrier_sem = pltpu.get_barrier_semaphore()
        for nbr in (left, right, diag):
            pl.semaphore_signal(
                barrier_sem, inc=1,
                device_id=(nbr,), device_id_type=pl.DeviceIdType.MESH,
            )
        pl.semaphore_wait(barrier_sem, 3)

        def copy(src, dst, idx, dev):
            return pltpu.make_async_remote_copy(
                src_ref=src, dst_ref=dst,
                send_sem=send_sems.at[idx], recv_sem=recv_sems.at[idx],
                device_id=(dev,), device_id_type=pl.DeviceIdType.MESH,
            )

        x_dma.wait()
        xs_ref[0, :, :] = x_vmem[0, :, :].astype(jnp.bfloat16)
        agr0 = copy(xs_ref.at[0], xs_ref.at[1], AGR0, right)
        agl0 = copy(xs_ref.at[0], xs_ref.at[3], AGL0, left)
        agr0.start()
        agl0.start()

        w_dmas[0].wait()
        wk = w_vmem[0, :, :].astype(jnp.bfloat16)
        w_dmas[1].wait()
        wv = w_vmem[1, :, :].astype(jnp.bfloat16)
        w_dmas[2].wait()
        wq = w_vmem[2, :, :].astype(jnp.bfloat16)
        w_dmas[3].wait()
        wo = w_vmem[3, :, :].astype(jnp.bfloat16)

        def kv_for(xb):
            k = lax.dot_general(xb, wk, (((1,), (0,)), ((), ())),
                                preferred_element_type=jnp.float32).astype(jnp.bfloat16)
            v = lax.dot_general(xb, wv, (((1,), (0,)), ((), ())),
                                preferred_element_type=jnp.float32).astype(jnp.bfloat16)
            return k, v

        def p_rows(xb_rows, k, v):
            q = lax.dot_general(xb_rows, wq, (((1,), (0,)), ((), ())),
                                preferred_element_type=jnp.float32).astype(jnp.bfloat16)
            cols = []
            for h in range(H):
                sl = slice(h * DH, (h + 1) * DH)
                s = lax.dot_general(q[:, sl], k[:, sl],
                                    (((1,), (1,)), ((), ())),
                                    preferred_element_type=jnp.float32)
                s = s * SCALE
                m = jnp.max(s, axis=1, keepdims=True)
                p = jnp.exp(s - m)
                p = p / jnp.sum(p, axis=1, keepdims=True)
                o = lax.dot_general(p.astype(jnp.bfloat16), v[:, sl],
                                    (((1,), (0,)), ((), ())),
                                    preferred_element_type=jnp.float32)
                cols.append(o)
            attn = jnp.concatenate(cols, axis=1).astype(jnp.bfloat16)
            return lax.dot_general(attn, wo, (((1,), (0,)), ((), ())),
                                   preferred_element_type=jnp.float32)

        def partial_for(xb):
            k, v = kv_for(xb)
            return p_rows(xb, k, v)

        pown_ref[:, :] = partial_for(xs_ref[0, :, :])

        agr0.wait_recv()
        agr1 = copy(xs_ref.at[1], xs_ref.at[2], AGR1, right)
        agr1.start()

        pl_s[:, :] = partial_for(xs_ref[1, :, :]).astype(jnp.bfloat16)
        pl_rdma = copy(pl_s, pl_r, PL, left)
        pl_rdma.start()

        agr1.wait_recv()
        k2, v2 = kv_for(xs_ref[2, :, :])
        pd_s[0, :, :] = p_rows(xs_ref[2, 0:HALF, :], k2, v2).astype(jnp.bfloat16)
        pda = copy(pd_s.at[0], pd_r.at[0], PDA, diag)
        pda.start()
        pd_s[1, :, :] = p_rows(xs_ref[2, HALF:SQ, :], k2, v2).astype(jnp.bfloat16)
        pdb = copy(pd_s.at[1], pd_r.at[1], PDB, diag)
        pdb.start()

        agl0.wait_recv()
        k3, v3 = kv_for(xs_ref[3, :, :])
        pr_s[0, :, :] = p_rows(xs_ref[3, 0:HALF, :], k3, v3).astype(jnp.bfloat16)
        pra = copy(pr_s.at[0], pr_r.at[0], PRA, right)
        pra.start()
        pr_s[1, :, :] = p_rows(xs_ref[3, HALF:SQ, :], k3, v3).astype(jnp.bfloat16)
        prb = copy(pr_s.at[1], pr_r.at[1], PRB, right)
        prb.start()

        pl_rdma.wait_recv()
        pda.wait_recv()
        pra.wait_recv()
        out_ref[0, 0:HALF, :] = (pown_ref[0:HALF, :]
                                 + pl_r[0:HALF, :].astype(jnp.float32)
                                 + pd_r[0, :, :].astype(jnp.float32)
                                 + pr_r[0, :, :].astype(jnp.float32))
        pdb.wait_recv()
        prb.wait_recv()
        out_ref[0, HALF:SQ, :] = (pown_ref[HALF:SQ, :]
                                  + pl_r[HALF:SQ, :].astype(jnp.float32)
                                  + pd_r[1, :, :].astype(jnp.float32)
                                  + pr_r[1, :, :].astype(jnp.float32))

        for rdma in (agr0, agl0, agr1, pl_rdma, pda, pdb, pra, prb):
            rdma.wait_send()

    return pl.pallas_call(
        body,
        out_shape=jax.ShapeDtypeStruct((1, SQ, D), jnp.float32),
        in_specs=[pl.BlockSpec(memory_space=pl.ANY)] * 5,
        out_specs=pl.BlockSpec(memory_space=pltpu.VMEM),
        scratch_shapes=[
            pltpu.VMEM((1, SQ, D), jnp.float32),
            pltpu.VMEM((N_DEV, SQ, D), jnp.bfloat16),
            pltpu.VMEM((SQ, D), jnp.float32),
            pltpu.VMEM((SQ, D), jnp.bfloat16),
            pltpu.VMEM((SQ, D), jnp.bfloat16),
            pltpu.VMEM((2, HALF, D), jnp.bfloat16),
            pltpu.VMEM((2, HALF, D), jnp.bfloat16),
            pltpu.VMEM((2, HALF, D), jnp.bfloat16),
            pltpu.VMEM((2, HALF, D), jnp.bfloat16),
            pltpu.VMEM((4, D, D), jnp.float32),
            pltpu.SemaphoreType.DMA((4,)),
            pltpu.SemaphoreType.DMA,
            pltpu.SemaphoreType.DMA((8,)),
            pltpu.SemaphoreType.DMA((8,)),
        ],
        compiler_params=pltpu.CompilerParams(
            collective_id=0, vmem_limit_bytes=64 * 1024 * 1024,
        ),
    )(x, Wq, Wo, Wk, Wv)


# device time: 40148 ns/iter; 1.7969x vs baseline; 1.0253x over previous
import jax
import jax.numpy as jnp
from jax import lax
from jax.experimental import pallas as pl
from jax.experimental.pallas import tpu as pltpu

N_DEV = 4
SQ = 256
HALF = SQ // 2
D = 1024
H = 8
DH = 128
SCALE = 0.08838834764831843

AGR0, AGR1, AGL0, PL, PDA, PDB, PRA, PRB = range(8)


def kernel(x, Wq, Wo, Wk, Wv):
    def body(x_ref, wq_ref, wo_ref, wk_ref, wv_ref, out_ref,
             x_vmem, xs_ref, pown_ref,
             pl_s, pl_r, pd_s, pd_r, pr_s, pr_r,
             w_vmem, w_sems, x_sem,
             send_sems, recv_sems):
        my = lax.axis_index("i")
        right = lax.rem(my + 1, N_DEV)
        left = lax.rem(my + N_DEV - 1, N_DEV)
        diag = lax.rem(my + 2, N_DEV)

        x_dma = pltpu.make_async_copy(x_ref, x_vmem, x_sem)
        x_dma.start()

        barrier_sem = pltpu.get_barrier_semaphore()
        for nbr in (left, right, diag):
            pl.semaphore_signal(
                barrier_sem, inc=1,
                device_id=(nbr,), device_id_type=pl.DeviceIdType.MESH,
            )
        pl.semaphore_wait(barrier_sem, 3)

        def copy(src, dst, idx, dev):
            return pltpu.make_async_remote_copy(
                src_ref=src, dst_ref=dst,
                send_sem=send_sems.at[idx], recv_sem=recv_sems.at[idx],
                device_id=(dev,), device_id_type=pl.DeviceIdType.MESH,
            )

        x_dma.wait()
        xs_ref[0, :, :] = x_vmem[0, :, :].astype(jnp.bfloat16)
        agr0 = copy(xs_ref.at[0], xs_ref.at[1], AGR0, right)
        agl0 = copy(xs_ref.at[0], xs_ref.at[3], AGL0, left)
        agr0.start()
        agl0.start()

        w_dmas = []
        for j, w_hbm in enumerate((wk_ref, wv_ref, wq_ref, wo_ref)):
            dma = pltpu.make_async_copy(w_hbm, w_vmem.at[j], w_sems.at[j])
            dma.start()
            w_dmas.append(dma)

        w_dmas[0].wait()
        wk = w_vmem[0, :, :].astype(jnp.bfloat16)
        w_dmas[1].wait()
        wv = w_vmem[1, :, :].astype(jnp.bfloat16)
        w_dmas[2].wait()
        wq = w_vmem[2, :, :].astype(jnp.bfloat16)
        w_dmas[3].wait()
        wo = w_vmem[3, :, :].astype(jnp.bfloat16)

        def kv_for(xb):
            k = lax.dot_general(xb, wk, (((1,), (0,)), ((), ())),
                                preferred_element_type=jnp.float32).astype(jnp.bfloat16)
            v = lax.dot_general(xb, wv, (((1,), (0,)), ((), ())),
                                preferred_element_type=jnp.float32).astype(jnp.bfloat16)
            return k, v

        def p_rows(xb_rows, k, v):
            q = lax.dot_general(xb_rows, wq, (((1,), (0,)), ((), ())),
                                preferred_element_type=jnp.float32).astype(jnp.bfloat16)
            cols = []
            for h in range(H):
                sl = slice(h * DH, (h + 1) * DH)
                s = lax.dot_general(q[:, sl], k[:, sl],
                                    (((1,), (1,)), ((), ())),
                                    preferred_element_type=jnp.float32)
                s = s * SCALE
                m = jnp.max(s, axis=1, keepdims=True)
                p = jnp.exp(s - m)
                p = p / jnp.sum(p, axis=1, keepdims=True)
                o = lax.dot_general(p.astype(jnp.bfloat16), v[:, sl],
                                    (((1,), (0,)), ((), ())),
                                    preferred_element_type=jnp.float32)
                cols.append(o)
            attn = jnp.concatenate(cols, axis=1).astype(jnp.bfloat16)
            return lax.dot_general(attn, wo, (((1,), (0,)), ((), ())),
                                   preferred_element_type=jnp.float32)

        def partial_for(xb):
            k, v = kv_for(xb)
            return p_rows(xb, k, v)

        pown_ref[:, :] = partial_for(xs_ref[0, :, :])

        agr0.wait_recv()
        agr1 = copy(xs_ref.at[1], xs_ref.at[2], AGR1, right)
        agr1.start()

        pl_s[:, :] = partial_for(xs_ref[1, :, :]).astype(jnp.bfloat16)
        pl_rdma = copy(pl_s, pl_r, PL, left)
        pl_rdma.start()

        agr1.wait_recv()
        k2, v2 = kv_for(xs_ref[2, :, :])
        pd_s[0, :, :] = p_rows(xs_ref[2, 0:HALF, :], k2, v2).astype(jnp.bfloat16)
        pda = copy(pd_s.at[0], pd_r.at[0], PDA, diag)
        pda.start()
        pd_s[1, :, :] = p_rows(xs_ref[2, HALF:SQ, :], k2, v2).astype(jnp.bfloat16)
        pdb = copy(pd_s.at[1], pd_r.at[1], PDB, diag)
        pdb.start()

        agl0.wait_recv()
        k3, v3 = kv_for(xs_ref[3, :, :])
        pr_s[0, :, :] = p_rows(xs_ref[3, 0:HALF, :], k3, v3).astype(jnp.bfloat16)
        pra = copy(pr_s.at[0], pr_r.at[0], PRA, right)
        pra.start()
        pr_s[1, :, :] = p_rows(xs_ref[3, HALF:SQ, :], k3, v3).astype(jnp.bfloat16)
        prb = copy(pr_s.at[1], pr_r.at[1], PRB, right)
        prb.start()

        pl_rdma.wait_recv()
        pda.wait_recv()
        pra.wait_recv()
        out_ref[0, 0:HALF, :] = (pown_ref[0:HALF, :]
                                 + pl_r[0:HALF, :].astype(jnp.float32)
                                 + pd_r[0, :, :].astype(jnp.float32)
                                 + pr_r[0, :, :].astype(jnp.float32))
        pdb.wait_recv()
        prb.wait_recv()
        out_ref[0, HALF:SQ, :] = (pown_ref[HALF:SQ, :]
                                  + pl_r[HALF:SQ, :].astype(jnp.float32)
                                  + pd_r[1, :, :].astype(jnp.float32)
                                  + pr_r[1, :, :].astype(jnp.float32))

        for rdma in (agr0, agl0, agr1, pl_rdma, pda, pdb, pra, prb):
            rdma.wait_send()

    return pl.pallas_call(
        body,
        out_shape=jax.ShapeDtypeStruct((1, SQ, D), jnp.float32),
        in_specs=[pl.BlockSpec(memory_space=pl.ANY)] * 5,
        out_specs=pl.BlockSpec(memory_space=pltpu.VMEM),
        scratch_shapes=[
            pltpu.VMEM((1, SQ, D), jnp.float32),
            pltpu.VMEM((N_DEV, SQ, D), jnp.bfloat16),
            pltpu.VMEM((SQ, D), jnp.float32),
            pltpu.VMEM((SQ, D), jnp.bfloat16),
            pltpu.VMEM((SQ, D), jnp.bfloat16),
            pltpu.VMEM((2, HALF, D), jnp.bfloat16),
            pltpu.VMEM((2, HALF, D), jnp.bfloat16),
            pltpu.VMEM((2, HALF, D), jnp.bfloat16),
            pltpu.VMEM((2, HALF, D), jnp.bfloat16),
            pltpu.VMEM((4, D, D), jnp.float32),
            pltpu.SemaphoreType.DMA((4,)),
            pltpu.SemaphoreType.DMA,
            pltpu.SemaphoreType.DMA((8,)),
            pltpu.SemaphoreType.DMA((8,)),
        ],
        compiler_params=pltpu.CompilerParams(
            collective_id=0, vmem_limit_bytes=64 * 1024 * 1024,
        ),
    )(x, Wq, Wo, Wk, Wv)
